# baseline (speedup 1.0000x reference)
import functools

import jax
import jax.numpy as jnp
from jax import lax
from jax.experimental import pallas as pl
from jax.experimental.pallas import tpu as pltpu
from jax.experimental.pallas import tpu_sc as plsc

D_MODEL = 128
SEQ_LEN = 512
VOCAB = 100
BATCH = 1024

NUM_CORES = 2
NUM_SUBCORES = 16
NUM_WORKERS = NUM_CORES * NUM_SUBCORES
SENT_PER_WORKER = BATCH // NUM_WORKERS
NCHUNK = 4
CHUNK = SEQ_LEN // NCHUNK


def _positional_encoding():
    index = jnp.arange(0, D_MODEL, 2).astype(jnp.float32)
    denominator = jnp.power(10000.0, index / D_MODEL)
    position = jnp.arange(SEQ_LEN, dtype=jnp.float32)[:, None]
    even = jnp.sin(position / denominator)
    odd = jnp.cos(position / denominator)
    return jnp.stack((even, odd), axis=2).reshape(SEQ_LEN, D_MODEL)


VPAD = 104


def _build_expanded_table(table_pad, pos):
    lblk = 32

    def body(tab_ref, pos_ref, o_ref):
        o_ref[...] = (
            pos_ref[...][:, None, :] + tab_ref[...][None, :, :]
        ).reshape(lblk * VPAD, D_MODEL)

    return pl.pallas_call(
        body,
        grid=(SEQ_LEN // lblk,),
        in_specs=[
            pl.BlockSpec((VPAD, D_MODEL), lambda i: (0, 0)),
            pl.BlockSpec((lblk, D_MODEL), lambda i: (i, 0)),
        ],
        out_specs=pl.BlockSpec((lblk * VPAD, D_MODEL), lambda i: (i, 0)),
        out_shape=jax.ShapeDtypeStruct((SEQ_LEN * VPAD, D_MODEL),
                                       jnp.float32),
    )(table_pad, pos)


NPHASE = 16
PCH = SEQ_LEN // NPHASE
ESP_ROWS = PCH * VPAD
NBUF = 4


def _sc_gather(x2, e2, offs):
    mesh = plsc.VectorSubcoreMesh(
        core_axis_name="c", subcore_axis_name="s",
        num_cores=NUM_CORES, num_subcores=NUM_SUBCORES)

    @functools.partial(
        pl.kernel,
        out_type=jax.ShapeDtypeStruct((BATCH, NPHASE, PCH, D_MODEL),
                                      jnp.float32),
        mesh=mesh,
        scratch_types=[
            pltpu.VMEM((SENT_PER_WORKER, SEQ_LEN), jnp.int32),
            pltpu.VMEM((NPHASE, SENT_PER_WORKER * PCH), jnp.int32),
            pltpu.VMEM((PCH,), jnp.int32),
            pltpu.VMEM((NBUF, 2 * PCH, D_MODEL), jnp.float32),
            pltpu.VMEM_SHARED((ESP_ROWS, D_MODEL), jnp.float32),
            pltpu.VMEM_SHARED((ESP_ROWS, D_MODEL), jnp.float32),
        ] + [pltpu.SemaphoreType.DMA] * (2 * NBUF + 2),
    )
    def k(x_ref, e_ref, offs_ref, out_ref, idx_all, comb2, offs_v, bufs,
          e_sp0, e_sp1, *sems):
        gsems = sems[:NBUF]
        ssems = sems[NBUF:2 * NBUF]
        stgsems = sems[2 * NBUF:]
        e_sps = (e_sp0, e_sp1)
        sid = lax.axis_index("s")
        wid = sid * NUM_CORES + lax.axis_index("c")
        base = wid * SENT_PER_WORKER

        @pl.when(sid == 0)
        def _stage01():
            pltpu.async_copy(
                e_ref.at[pl.ds(0, ESP_ROWS)], e_sps[0], stgsems[0])
            pltpu.async_copy(
                e_ref.at[pl.ds(ESP_ROWS, ESP_ROWS)], e_sps[1], stgsems[1])

        pltpu.sync_copy(offs_ref, offs_v)
        pltpu.sync_copy(x_ref.at[pl.ds(base, SENT_PER_WORKER)], idx_all)

        def combi(b, carry):
            for ph in range(NPHASE):
                for r in range(PCH // 16):
                    sl = pl.ds(r * 16, 16)
                    comb2[ph, pl.ds(b * PCH + r * 16, 16)] = (
                        idx_all[b, pl.ds(ph * PCH + r * 16, 16)]
                        + offs_v[sl])
            return carry

        lax.fori_loop(0, SENT_PER_WORKER, combi, 0)

        for ph in range(NPHASE):
            e_sp = e_sps[ph % 2]

            @pl.when(sid == 0)
            def _wait_stage():
                pltpu.make_async_copy(
                    e_ref.at[pl.ds(ph * ESP_ROWS, ESP_ROWS)], e_sp,
                    stgsems[ph % 2]).wait()

            plsc.subcore_barrier()

            def group(g, carry):
                for j in range(NBUF):
                    pr = g * NBUF + j
                    b0 = base + 2 * pr
                    def _wait_prev(b0=b0, j=j):
                        pltpu.make_async_copy(
                            bufs.at[j, pl.ds(0, PCH)],
                            out_ref.at[b0, ph], ssems[j]).wait()
                        pltpu.make_async_copy(
                            bufs.at[j, pl.ds(PCH, PCH)],
                            out_ref.at[b0 + 1, ph], ssems[j]).wait()
                    if ph == 0:
                        pl.when(g > 0)(_wait_prev)
                    else:
                        _wait_prev()
                    pltpu.async_copy(
                        e_sp.at[comb2.at[ph, pl.ds(pr * 2 * PCH, 2 * PCH)]],
                        bufs.at[j], gsems[j])
                for j in range(NBUF):
                    pr = g * NBUF + j
                    b0 = base + 2 * pr
                    pltpu.make_async_copy(
                        e_sp.at[comb2.at[ph, pl.ds(pr * 2 * PCH, 2 * PCH)]],
                        bufs.at[j], gsems[j]).wait()
                    pltpu.async_copy(bufs.at[j, pl.ds(0, PCH)],
                                     out_ref.at[b0, ph], ssems[j])
                    pltpu.async_copy(bufs.at[j, pl.ds(PCH, PCH)],
                                     out_ref.at[b0 + 1, ph], ssems[j])
                return carry

            lax.fori_loop(0, SENT_PER_WORKER // (2 * NBUF), group, 0)

            plsc.subcore_barrier()
            if ph + 2 < NPHASE:
                @pl.when(sid == 0)
                def _stage_next():
                    pltpu.async_copy(
                        e_ref.at[pl.ds((ph + 2) * ESP_ROWS, ESP_ROWS)],
                        e_sps[ph % 2], stgsems[ph % 2])

        for j in range(NBUF):
            b0 = base + SENT_PER_WORKER - 2 * NBUF + 2 * j
            pltpu.make_async_copy(
                bufs.at[j, pl.ds(0, PCH)],
                out_ref.at[b0, NPHASE - 1], ssems[j]).wait()
            pltpu.make_async_copy(
                bufs.at[j, pl.ds(PCH, PCH)],
                out_ref.at[b0 + 1, NPHASE - 1], ssems[j]).wait()

    return k(x2, e2, offs)


def kernel(x, table):
    pos = _positional_encoding()
    table_pad = jnp.pad(table, ((0, VPAD - VOCAB), (0, 0)))
    e2 = _build_expanded_table(table_pad, pos)
    x2 = x.astype(jnp.int32)
    offs = jnp.arange(PCH, dtype=jnp.int32) * VPAD
    out4 = _sc_gather(x2, e2, offs)
    return out4.reshape(BATCH, SEQ_LEN, D_MODEL)
